# baseline (speedup 1.0000x reference)
import functools

import jax
import jax.numpy as jnp
from jax import lax
from jax.experimental import pallas as pl
from jax.experimental.pallas import tpu as pltpu
from jax.experimental.pallas import tpu_sc as plsc

_NUM_OBJS = 1000
_NUM_RELS = 64
_BATCH = 16384

_INFO = plsc.get_sparse_core_info()
_NC = _INFO.num_cores
_NS = _INFO.num_subcores
_NW = _NC * _NS
_L = _INFO.num_lanes

_BPW = _BATCH // _NW
_CHUNK = 128
_NCHUNK = _BPW // _CHUNK
_W2 = 2 * _NUM_RELS


def _make_kernel():
    mesh = plsc.VectorSubcoreMesh(core_axis_name="c", subcore_axis_name="s")

    @functools.partial(
        pl.kernel,
        mesh=mesh,
        compiler_params=pltpu.CompilerParams(needs_layout_passes=False),
        out_type=jax.ShapeDtypeStruct((_BATCH, _NUM_RELS), jnp.float32),
        scratch_types=[
            pltpu.VMEM((_BPW,), jnp.int32),
            pltpu.VMEM((_BPW,), jnp.int32),
            pltpu.VMEM((_NCHUNK, _CHUNK), jnp.int32),
            pltpu.VMEM((_BPW,), jnp.int32),
            pltpu.VMEM((_CHUNK, _W2), jnp.float32),
            pltpu.VMEM((_CHUNK, _W2), jnp.float32),
            pltpu.VMEM((_CHUNK, _NUM_RELS), jnp.float32),
            pltpu.VMEM((_CHUNK, _NUM_RELS), jnp.float32),
            pltpu.SemaphoreType.DMA,
            pltpu.SemaphoreType.DMA,
        ],
    )
    def gather_kernel(l0_hbm, l1_hbm, table_hbm, out_hbm,
                      l0_v, l1_v, idx_v, off_v, rows_a, rows_b,
                      out_a, out_b, gsem, osem):
        wid = lax.axis_index("s") * _NC + lax.axis_index("c")
        base = wid * _BPW

        pltpu.sync_copy(l0_hbm.at[pl.ds(base, _BPW)], l0_v)
        pltpu.sync_copy(l1_hbm.at[pl.ds(base, _BPW)], l1_v)

        for c in range(_NCHUNK):
            for k in range(_CHUNK // _L):
                j = c * (_CHUNK // _L) + k
                s = pl.ds(j * _L, _L)
                l1 = l1_v[s]
                idx_v[c, pl.ds(k * _L, _L)] = (
                    l0_v[s] * (_NUM_OBJS // 2)
                    + lax.shift_right_logical(l1, 1))
                off_v[s] = lax.shift_left(l1 & 1, 6)

        rows_bufs = [rows_a, rows_b]
        out_bufs = [out_a, out_b]
        lane = lax.iota(jnp.int32, _L)

        def fire_gather(c):
            return pltpu.async_copy(
                table_hbm.at[idx_v.at[c]], rows_bufs[c % 2], gsem)

        gathers = [fire_gather(0)]
        out_copies = []
        for c in range(_NCHUNK):
            if c + 1 < _NCHUNK:
                gathers.append(fire_gather(c + 1))
            gathers[c].wait()
            if c >= 2:
                out_copies[c - 2].wait()

            rows = rows_bufs[c % 2]
            outb = out_bufs[c % 2]

            def select_rows(m, _, c=c, rows=rows, outb=outb):
                r_vec = m * _L + lane
                p64 = off_v[pl.ds(c * _CHUNK + m * _L, _L)]
                for col in range(_NUM_RELS):
                    col_v = jnp.full((_L,), col, jnp.int32)
                    val = plsc.load_gather(rows, [r_vec, p64 + col_v])
                    plsc.store_scatter(outb, [r_vec, col_v], val)
                return _

            lax.fori_loop(0, _CHUNK // _L, select_rows, None)
            out_copies.append(
                pltpu.async_copy(
                    outb, out_hbm.at[pl.ds(base + c * _CHUNK, _CHUNK)], osem))
        for cp in out_copies[-2:]:
            cp.wait()

    return gather_kernel


_GATHER = _make_kernel()


@jax.jit
def kernel(labels, obj_baseline):
    table2 = obj_baseline.reshape(_NUM_OBJS * _NUM_OBJS // 2, _W2)
    return _GATHER(labels[:, 0], labels[:, 1], table2)
